# baseline (speedup 1.0000x reference)
import functools

import jax
import jax.numpy as jnp
from jax import lax
from jax.experimental import pallas as pl
from jax.experimental.pallas import tpu as pltpu
from jax.experimental.pallas import tpu_sc as plsc

NN = 10000
EE = 320000
D = 128
NP = 10240
NC, NS = 2, 16
NW = NC * NS
CH = 32
NCHUNK = 320
EPT = CH * NCHUNK
EP = EPT * NW
DW = 128
RB = 1024
GRID = NP // RB


NB = 8
NTL = 8


def _make_edge_pass(with_deg):
    mesh = plsc.VectorSubcoreMesh(core_axis_name="c", subcore_axis_name="s")
    rpt = NP // NS
    QC = NCHUNK // NTL
    NG = QC // NB

    out_type = [jax.ShapeDtypeStruct((NC, NP, D), jnp.float32)]
    if with_deg:
        out_type.append(jax.ShapeDtypeStruct((NC, NP, D), jnp.float32))
    scratch = (
        [pltpu.VMEM((QC, CH), jnp.int32)] * 2 +
        [pltpu.VMEM((CH, D), jnp.float32)] * NB +
        [pltpu.SemaphoreType.DMA] * NB +
        [pltpu.SemaphoreType.DMA] * NB +
        [pltpu.VMEM_SHARED((NP, D), jnp.float32)]
    )

    def body(y_hbm, src_hbm, dst_hbm, zrows_hbm, ones_hbm, *rest):
        if with_deg:
            acc_out, deg_out = rest[0], rest[1]
            rest = rest[2:]
        else:
            acc_out = rest[0]
            rest = rest[1:]
        src_v, dst_v = rest[0], rest[1]
        rest = rest[2:]
        rows = rest[:NB]
        gsem = rest[NB:2 * NB]
        ssem = rest[2 * NB:3 * NB]
        acc_sh = rest[3 * NB]
        c = lax.axis_index("c")
        s = lax.axis_index("s")
        wid = s * NC + c

        pltpu.sync_copy(zrows_hbm, acc_sh.at[pl.ds(s * rpt, rpt)])

        if with_deg:
            pltpu.sync_copy(ones_hbm, rows[0])
            plsc.subcore_barrier()

            def dstart(j, t):
                pltpu.async_copy(rows[0], acc_sh.at[dst_v.at[j]], ssem[t],
                                 add=True)

            def dwait(t):
                pltpu.make_async_copy(rows[0], acc_sh.at[dst_v.at[0]],
                                      ssem[t]).wait()

            for h in range(NTL):
                pltpu.sync_copy(dst_hbm.at[wid, pl.ds(h * QC, QC)], dst_v)
                for t in range(NB):
                    dstart(t, t)

                def dgroup(i, _):
                    k = i * NB
                    for t in range(NB):
                        dwait(t)
                        dstart(k + NB + t, t)
                    return 0

                lax.fori_loop(0, NG - 1, dgroup, 0)
                for t in range(NB):
                    dwait(t)

            plsc.subcore_barrier()
            pltpu.sync_copy(acc_sh.at[pl.ds(s * rpt, rpt)],
                            deg_out.at[c, pl.ds(s * rpt, rpt)])
            pltpu.sync_copy(zrows_hbm, acc_sh.at[pl.ds(s * rpt, rpt)])

        def g_start(j, t):
            pltpu.async_copy(y_hbm.at[src_v.at[j]], rows[t], gsem[t])

        def g_wait(j, t):
            pltpu.make_async_copy(y_hbm.at[src_v.at[j]], rows[t],
                                  gsem[t]).wait()

        def s_start(j, t):
            pltpu.async_copy(rows[t], acc_sh.at[dst_v.at[j]], ssem[t],
                             add=True)

        def s_wait(j, t):
            pltpu.make_async_copy(rows[t], acc_sh.at[dst_v.at[j]],
                                  ssem[t]).wait()

        for h in range(NTL):
            pltpu.sync_copy(src_hbm.at[wid, pl.ds(h * QC, QC)], src_v)
            pltpu.sync_copy(dst_hbm.at[wid, pl.ds(h * QC, QC)], dst_v)
            if h == 0:
                plsc.subcore_barrier()
            for t in range(NB):
                g_start(t, t)

            def group(i, _):
                k = i * NB
                for t in range(NB):
                    g_wait(k + t, t)
                    s_start(k + t, t)
                for t in range(NB):
                    s_wait(k + t, t)
                    g_start(k + NB + t, t)
                return 0

            lax.fori_loop(0, NG - 1, group, 0)
            k = (NG - 1) * NB
            for t in range(NB):
                g_wait(k + t, t)
                s_start(k + t, t)
            for t in range(NB):
                s_wait(k + t, t)

        plsc.subcore_barrier()
        pltpu.sync_copy(acc_sh.at[pl.ds(s * rpt, rpt)],
                        acc_out.at[c, pl.ds(s * rpt, rpt)])

    return pl.kernel(body, out_type=out_type, mesh=mesh,
                     scratch_types=scratch)


@functools.lru_cache(maxsize=None)
def _edge_pass_cached(with_deg):
    return _make_edge_pass(with_deg)


def _edge_pass(*args):
    return _edge_pass_cached(False)(*args)


def _edge_pass_deg(*args):
    return _edge_pass_cached(True)(*args)


def _proj_body(x_ref, w_ref, o_ref):
    o_ref[...] = jnp.dot(x_ref[...], w_ref[...],
                         preferred_element_type=jnp.float32)


def _rdeg(deg_ref):
    d = deg_ref[0, :, 0:1] + deg_ref[1, :, 0:1]
    return 1.0 / jnp.maximum(d, 1.0)


def _stage_b_body(x_ref, acc_ref, deg_ref, ws_ref, b_ref, wn2_ref,
                  h1_ref, y2_ref):
    hn = (acc_ref[0] + acc_ref[1]) * _rdeg(deg_ref)
    h1 = jnp.maximum(
        jnp.dot(x_ref[...], ws_ref[...], preferred_element_type=jnp.float32)
        + hn + b_ref[...], 0.0)
    h1_ref[...] = h1
    y2_ref[...] = jnp.dot(h1, wn2_ref[...],
                          preferred_element_type=jnp.float32)


def _stage_c_body(h1_ref, acc_ref, deg_ref, ws2_ref, b2_ref, wm1_ref,
                  bm1_ref, wm2_ref, bm2_ref, o_ref):
    hn = (acc_ref[0] + acc_ref[1]) * _rdeg(deg_ref)
    h2 = jnp.maximum(
        jnp.dot(h1_ref[...], ws2_ref[...], preferred_element_type=jnp.float32)
        + hn + b2_ref[...], 0.0)
    m = jnp.maximum(
        jnp.dot(h2, wm1_ref[...], preferred_element_type=jnp.float32)
        + bm1_ref[...], 0.0)
    o_ref[...] = (jnp.dot(m, wm2_ref[...], preferred_element_type=jnp.float32)
                  + bm2_ref[...])


_row_spec = pl.BlockSpec((RB, D), lambda j: (j, 0))
_acc_spec = pl.BlockSpec((NC, RB, D), lambda j: (0, j, 0))
_deg_spec = pl.BlockSpec((NC, RB, DW), lambda j: (0, j, 0))
_w_spec = pl.BlockSpec((D, D), lambda j: (0, 0))
_b_spec = pl.BlockSpec((1, D), lambda j: (0, 0))


def _proj(x, w):
    return pl.pallas_call(
        _proj_body, grid=(GRID,),
        in_specs=[_row_spec, _w_spec], out_specs=_row_spec,
        out_shape=jax.ShapeDtypeStruct((NP, D), jnp.float32),
    )(x, w)


def _stage_b(x, acc, deg, ws, b, wn2):
    return pl.pallas_call(
        _stage_b_body, grid=(GRID,),
        in_specs=[_row_spec, _acc_spec, _deg_spec, _w_spec, _b_spec, _w_spec],
        out_specs=[_row_spec, _row_spec],
        out_shape=[jax.ShapeDtypeStruct((NP, D), jnp.float32),
                   jax.ShapeDtypeStruct((NP, D), jnp.float32)],
    )(x, acc, deg, ws, b, wn2)


def _stage_c(h1, acc, deg, ws2, b2, wm1, bm1, wm2, bm2):
    return pl.pallas_call(
        _stage_c_body, grid=(GRID,),
        in_specs=[_row_spec, _acc_spec, _deg_spec, _w_spec, _b_spec,
                  _w_spec, _b_spec, _w_spec, _b_spec],
        out_specs=_row_spec,
        out_shape=jax.ShapeDtypeStruct((NP, D), jnp.float32),
    )(h1, acc, deg, ws2, b2, wm1, bm1, wm2, bm2)


def kernel(features, edge_index, W_self1, W_neigh1, b1, W_self2, W_neigh2,
           b2, W_m1, b_m1, W_m2, b_m2):
    f32 = jnp.float32
    x_pad = jnp.zeros((NP, D), f32).at[:NN, :].set(features)
    pad = jnp.full((EP - EE,), NP - 1, jnp.int32)
    src = jnp.concatenate([edge_index[0], pad]).reshape(NW, NCHUNK, CH)
    dst = jnp.concatenate([edge_index[1], pad]).reshape(NW, NCHUNK, CH)

    zrows = jnp.zeros((NP // NS, D), f32)
    ones = jnp.ones((CH, D), f32)

    b1r = b1.reshape(1, D)
    b2r = b2.reshape(1, D)
    bm1r = b_m1.reshape(1, D)
    bm2r = b_m2.reshape(1, D)

    y1 = _proj(x_pad, W_neigh1)
    acc1, deg = _edge_pass_deg(y1, src, dst, zrows, ones)
    h1, y2 = _stage_b(x_pad, acc1, deg, W_self1, b1r, W_neigh2)
    (acc2,) = _edge_pass(y2, src, dst, zrows, ones)
    out = _stage_c(h1, acc2, deg, W_self2, b2r, W_m1, bm1r, W_m2, bm2r)
    return out[:NN]


# speedup vs baseline: 1.0351x; 1.0351x over previous
import functools

import jax
import jax.numpy as jnp
from jax import lax
from jax.experimental import pallas as pl
from jax.experimental.pallas import tpu as pltpu
from jax.experimental.pallas import tpu_sc as plsc

NN = 10000
EE = 320000
D = 128
NP = 10240
NC, NS = 2, 16
NW = NC * NS
CH = 64
NCHUNK = 160
EPT = CH * NCHUNK
EP = EPT * NW
DW = 128
RB = 1024
GRID = NP // RB


NB = 4
NTL = 4


def _make_edge_pass(with_deg):
    mesh = plsc.VectorSubcoreMesh(core_axis_name="c", subcore_axis_name="s")
    rpt = NP // NS
    QC = NCHUNK // NTL
    NG = QC // NB

    out_type = [jax.ShapeDtypeStruct((NC, NP, D), jnp.float32)]
    if with_deg:
        out_type.append(jax.ShapeDtypeStruct((NC, NP, D), jnp.float32))
    scratch = (
        [pltpu.VMEM((QC, CH), jnp.int32)] * 2 +
        [pltpu.VMEM((CH, D), jnp.float32)] * NB +
        [pltpu.SemaphoreType.DMA] * NB +
        [pltpu.SemaphoreType.DMA] * NB +
        [pltpu.VMEM_SHARED((NP, D), jnp.float32)]
    )

    def body(y_hbm, src_hbm, dst_hbm, zrows_hbm, ones_hbm, *rest):
        if with_deg:
            acc_out, deg_out = rest[0], rest[1]
            rest = rest[2:]
        else:
            acc_out = rest[0]
            rest = rest[1:]
        src_v, dst_v = rest[0], rest[1]
        rest = rest[2:]
        rows = rest[:NB]
        gsem = rest[NB:2 * NB]
        ssem = rest[2 * NB:3 * NB]
        acc_sh = rest[3 * NB]
        c = lax.axis_index("c")
        s = lax.axis_index("s")
        wid = s * NC + c

        pltpu.sync_copy(zrows_hbm, acc_sh.at[pl.ds(s * rpt, rpt)])

        if with_deg:
            pltpu.sync_copy(ones_hbm, rows[0])
            plsc.subcore_barrier()

            def dstart(j, t):
                pltpu.async_copy(rows[0], acc_sh.at[dst_v.at[j]], ssem[t],
                                 add=True)

            def dwait(t):
                pltpu.make_async_copy(rows[0], acc_sh.at[dst_v.at[0]],
                                      ssem[t]).wait()

            for h in range(NTL):
                pltpu.sync_copy(dst_hbm.at[wid, pl.ds(h * QC, QC)], dst_v)
                for t in range(NB):
                    dstart(t, t)

                def dgroup(i, _):
                    k = i * NB
                    for t in range(NB):
                        dwait(t)
                        dstart(k + NB + t, t)
                    return 0

                lax.fori_loop(0, NG - 1, dgroup, 0)
                for t in range(NB):
                    dwait(t)

            plsc.subcore_barrier()
            pltpu.sync_copy(acc_sh.at[pl.ds(s * rpt, rpt)],
                            deg_out.at[c, pl.ds(s * rpt, rpt)])
            pltpu.sync_copy(zrows_hbm, acc_sh.at[pl.ds(s * rpt, rpt)])

        def g_start(j, t):
            pltpu.async_copy(y_hbm.at[src_v.at[j]], rows[t], gsem[t])

        def g_wait(j, t):
            pltpu.make_async_copy(y_hbm.at[src_v.at[j]], rows[t],
                                  gsem[t]).wait()

        def s_start(j, t):
            pltpu.async_copy(rows[t], acc_sh.at[dst_v.at[j]], ssem[t],
                             add=True)

        def s_wait(j, t):
            pltpu.make_async_copy(rows[t], acc_sh.at[dst_v.at[j]],
                                  ssem[t]).wait()

        for h in range(NTL):
            pltpu.sync_copy(src_hbm.at[wid, pl.ds(h * QC, QC)], src_v)
            pltpu.sync_copy(dst_hbm.at[wid, pl.ds(h * QC, QC)], dst_v)
            if h == 0:
                plsc.subcore_barrier()
            for t in range(NB):
                g_start(t, t)

            def group(i, _):
                k = i * NB
                for t in range(NB):
                    g_wait(k + t, t)
                    s_start(k + t, t)
                for t in range(NB):
                    s_wait(k + t, t)
                    g_start(k + NB + t, t)
                return 0

            lax.fori_loop(0, NG - 1, group, 0)
            k = (NG - 1) * NB
            for t in range(NB):
                g_wait(k + t, t)
                s_start(k + t, t)
            for t in range(NB):
                s_wait(k + t, t)

        plsc.subcore_barrier()
        pltpu.sync_copy(acc_sh.at[pl.ds(s * rpt, rpt)],
                        acc_out.at[c, pl.ds(s * rpt, rpt)])

    return pl.kernel(body, out_type=out_type, mesh=mesh,
                     scratch_types=scratch)


@functools.lru_cache(maxsize=None)
def _edge_pass_cached(with_deg):
    return _make_edge_pass(with_deg)


def _edge_pass(*args):
    return _edge_pass_cached(False)(*args)


def _edge_pass_deg(*args):
    return _edge_pass_cached(True)(*args)


def _proj_body(x_ref, w_ref, o_ref):
    o_ref[...] = jnp.dot(x_ref[...], w_ref[...],
                         preferred_element_type=jnp.float32)


def _rdeg(deg_ref):
    d = deg_ref[0, :, 0:1] + deg_ref[1, :, 0:1]
    return 1.0 / jnp.maximum(d, 1.0)


def _stage_b_body(x_ref, acc_ref, deg_ref, ws_ref, b_ref, wn2_ref,
                  h1_ref, y2_ref):
    hn = (acc_ref[0] + acc_ref[1]) * _rdeg(deg_ref)
    h1 = jnp.maximum(
        jnp.dot(x_ref[...], ws_ref[...], preferred_element_type=jnp.float32)
        + hn + b_ref[...], 0.0)
    h1_ref[...] = h1
    y2_ref[...] = jnp.dot(h1, wn2_ref[...],
                          preferred_element_type=jnp.float32)


def _stage_c_body(h1_ref, acc_ref, deg_ref, ws2_ref, b2_ref, wm1_ref,
                  bm1_ref, wm2_ref, bm2_ref, o_ref):
    hn = (acc_ref[0] + acc_ref[1]) * _rdeg(deg_ref)
    h2 = jnp.maximum(
        jnp.dot(h1_ref[...], ws2_ref[...], preferred_element_type=jnp.float32)
        + hn + b2_ref[...], 0.0)
    m = jnp.maximum(
        jnp.dot(h2, wm1_ref[...], preferred_element_type=jnp.float32)
        + bm1_ref[...], 0.0)
    o_ref[...] = (jnp.dot(m, wm2_ref[...], preferred_element_type=jnp.float32)
                  + bm2_ref[...])


_row_spec = pl.BlockSpec((RB, D), lambda j: (j, 0))
_acc_spec = pl.BlockSpec((NC, RB, D), lambda j: (0, j, 0))
_deg_spec = pl.BlockSpec((NC, RB, DW), lambda j: (0, j, 0))
_w_spec = pl.BlockSpec((D, D), lambda j: (0, 0))
_b_spec = pl.BlockSpec((1, D), lambda j: (0, 0))


def _proj(x, w):
    return pl.pallas_call(
        _proj_body, grid=(GRID,),
        in_specs=[_row_spec, _w_spec], out_specs=_row_spec,
        out_shape=jax.ShapeDtypeStruct((NP, D), jnp.float32),
    )(x, w)


def _stage_b(x, acc, deg, ws, b, wn2):
    return pl.pallas_call(
        _stage_b_body, grid=(GRID,),
        in_specs=[_row_spec, _acc_spec, _deg_spec, _w_spec, _b_spec, _w_spec],
        out_specs=[_row_spec, _row_spec],
        out_shape=[jax.ShapeDtypeStruct((NP, D), jnp.float32),
                   jax.ShapeDtypeStruct((NP, D), jnp.float32)],
    )(x, acc, deg, ws, b, wn2)


def _stage_c(h1, acc, deg, ws2, b2, wm1, bm1, wm2, bm2):
    return pl.pallas_call(
        _stage_c_body, grid=(GRID,),
        in_specs=[_row_spec, _acc_spec, _deg_spec, _w_spec, _b_spec,
                  _w_spec, _b_spec, _w_spec, _b_spec],
        out_specs=_row_spec,
        out_shape=jax.ShapeDtypeStruct((NP, D), jnp.float32),
    )(h1, acc, deg, ws2, b2, wm1, bm1, wm2, bm2)


def kernel(features, edge_index, W_self1, W_neigh1, b1, W_self2, W_neigh2,
           b2, W_m1, b_m1, W_m2, b_m2):
    f32 = jnp.float32
    x_pad = jnp.zeros((NP, D), f32).at[:NN, :].set(features)
    pad = jnp.full((EP - EE,), NP - 1, jnp.int32)
    src = jnp.concatenate([edge_index[0], pad]).reshape(NW, NCHUNK, CH)
    dst = jnp.concatenate([edge_index[1], pad]).reshape(NW, NCHUNK, CH)

    zrows = jnp.zeros((NP // NS, D), f32)
    ones = jnp.ones((CH, D), f32)

    b1r = b1.reshape(1, D)
    b2r = b2.reshape(1, D)
    bm1r = b_m1.reshape(1, D)
    bm2r = b_m2.reshape(1, D)

    y1 = _proj(x_pad, W_neigh1)
    acc1, deg = _edge_pass_deg(y1, src, dst, zrows, ones)
    h1, y2 = _stage_b(x_pad, acc1, deg, W_self1, b1r, W_neigh2)
    (acc2,) = _edge_pass(y2, src, dst, zrows, ones)
    out = _stage_c(h1, acc2, deg, W_self2, b2r, W_m1, bm1r, W_m2, bm2r)
    return out[:NN]
